# baseline (speedup 1.0000x reference)
import functools

import jax
import jax.numpy as jnp
from jax import lax
from jax.experimental import pallas as pl
from jax.experimental.pallas import tpu as pltpu
from jax.experimental.pallas import tpu_sc as plsc

E = 8
TOPK = 2
T = 8192
D_MODEL = 2048
D_FF = 1408
BLOCK = 128

TILE = 256
N_PAD = T * TOPK + E * TILE
NT = N_PAD // TILE


def _dequant_body(w_ref, s_ref, o_ref):
    j = pl.program_id(1)
    w = w_ref[0]
    s = s_ref[0, pl.ds(j, 1), :][0]
    ncols = w.shape[1]
    wr = w.reshape(BLOCK, ncols // BLOCK, BLOCK)
    o = wr * s[None, :, None]
    o_ref[0] = o.reshape(BLOCK, ncols).astype(jnp.bfloat16)


def _dequant_to_bf16(w, s):
    _, n, k = w.shape
    return pl.pallas_call(
        _dequant_body,
        grid=(E, n // BLOCK),
        in_specs=[
            pl.BlockSpec((1, BLOCK, k), lambda e, j: (e, j, 0)),
            pl.BlockSpec((1, n // BLOCK, k // BLOCK), lambda e, j: (e, 0, 0)),
        ],
        out_specs=pl.BlockSpec((1, BLOCK, k), lambda e, j: (e, j, 0)),
        out_shape=jax.ShapeDtypeStruct(w.shape, jnp.bfloat16),
    )(w, s)


_NW = 32
_RPW = N_PAD // _NW
_CH = 24
_NCHUNK = _RPW // _CH
_SL = D_MODEL // 128


def _sc_gather_body(x_hbm, idx_hbm, out_hbm, idx_v, rows0, rows1, s0, s1):
    wid = lax.axis_index("s") * 2 + lax.axis_index("c")
    pltpu.sync_copy(idx_hbm.at[wid], idx_v)

    bufs = (rows0, rows1)
    sems = (s0, s1)
    copies = []
    for c in range(_NCHUNK):
        copies.append(pltpu.async_copy(
            x_hbm.at[idx_v.at[c]], bufs[c % 2], sems[c % 2]))
        if c >= 1:
            copies[c - 1].wait()
            pltpu.sync_copy(
                bufs[(c - 1) % 2],
                out_hbm.at[pl.ds(wid * _RPW + (c - 1) * _CH, _CH)])
    copies[-1].wait()
    pltpu.sync_copy(
        bufs[(_NCHUNK - 1) % 2],
        out_hbm.at[pl.ds(wid * _RPW + (_NCHUNK - 1) * _CH, _CH)])


def _sc_gather(x2, idx3):
    mesh = plsc.VectorSubcoreMesh(core_axis_name="c", subcore_axis_name="s")
    f = functools.partial(
        pl.kernel,
        mesh=mesh,
        out_type=jax.ShapeDtypeStruct((N_PAD, D_MODEL), jnp.float32),
        scratch_types=[
            pltpu.VMEM((_NCHUNK, _CH), jnp.int32),
            pltpu.VMEM((_CH, D_MODEL), jnp.float32),
            pltpu.VMEM((_CH, D_MODEL), jnp.float32),
            pltpu.SemaphoreType.DMA,
            pltpu.SemaphoreType.DMA,
        ],
    )(_sc_gather_body)
    return f(x2, idx3)


def _ffn_body(te_ref, xs_ref, w1_ref, w3_ref, w2_ref, rw_ref, ys_ref):
    i = pl.program_id(0)

    @pl.when(te_ref[i] < E)
    def _():
        xb = xs_ref[...].astype(jnp.bfloat16)
        dn = (((1,), (1,)), ((), ()))
        g = jax.lax.dot_general(xb, w1_ref[0], dn,
                                preferred_element_type=jnp.float32)
        u = jax.lax.dot_general(xb, w3_ref[0], dn,
                                preferred_element_type=jnp.float32)
        h = (g / (1.0 + jnp.exp(-g)) * u).astype(jnp.bfloat16)
        y = jax.lax.dot_general(h, w2_ref[0], dn,
                                preferred_element_type=jnp.float32)
        ys_ref[...] = (y * rw_ref[0, 0, :][:, None]).astype(jnp.bfloat16)


def _grouped_ffn(tile_expert, xs, w13b, w2b, row_w3d):
    grid_spec = pltpu.PrefetchScalarGridSpec(
        num_scalar_prefetch=1,
        grid=(NT,),
        in_specs=[
            pl.BlockSpec((TILE, D_MODEL), lambda i, te: (i, 0)),
            pl.BlockSpec((1, D_FF, D_MODEL),
                         lambda i, te: (jnp.minimum(te[i], E - 1), 0, 0)),
            pl.BlockSpec((1, D_FF, D_MODEL),
                         lambda i, te: (jnp.minimum(te[i], E - 1), 1, 0)),
            pl.BlockSpec((1, D_MODEL, D_FF),
                         lambda i, te: (jnp.minimum(te[i], E - 1), 0, 0)),
            pl.BlockSpec((1, 1, TILE), lambda i, te: (i, 0, 0)),
        ],
        out_specs=pl.BlockSpec((TILE, D_MODEL), lambda i, te: (i, 0)),
    )
    return pl.pallas_call(
        _ffn_body,
        grid_spec=grid_spec,
        out_shape=jax.ShapeDtypeStruct((N_PAD, D_MODEL), jnp.bfloat16),
        compiler_params=pltpu.CompilerParams(
            dimension_semantics=("arbitrary",)),
    )(tile_expert, xs, w13b, w13b, w2b, row_w3d)


def kernel(x, topk_ids, topk_weights, w13, w13_scale_inv, w2, w2_scale_inv):
    ids = topk_ids.reshape(-1).astype(jnp.int32)
    wflat = topk_weights.reshape(-1).astype(jnp.float32)
    tok = (jnp.arange(T * TOPK, dtype=jnp.int32) // TOPK)

    oh = (ids[:, None] == jnp.arange(E, dtype=jnp.int32)[None, :])
    oh = oh.astype(jnp.int32)
    incl = lax.associative_scan(jnp.add, oh, axis=0)
    rank = jnp.sum((incl - oh) * oh, axis=1)
    counts = incl[-1]
    padded = ((counts + TILE - 1) // TILE) * TILE
    pad_off = jnp.cumsum(padded) - padded
    pos = pad_off[ids] + rank

    row_token = jnp.zeros((N_PAD,), jnp.int32).at[pos].set(tok)
    row_w = jnp.zeros((N_PAD,), jnp.float32).at[pos].set(wflat)

    tile_start = jnp.arange(NT, dtype=jnp.int32) * TILE
    pad_end = jnp.cumsum(padded)
    tile_expert = jnp.searchsorted(pad_end, tile_start, side="right")
    tile_expert = tile_expert.astype(jnp.int32)

    w13b = _dequant_to_bf16(w13, w13_scale_inv)
    w2b = _dequant_to_bf16(w2, w2_scale_inv)

    xs = _sc_gather(x, row_token.reshape(_NW, _NCHUNK, _CH))

    ys = _grouped_ffn(tile_expert, xs, w13b, w2b,
                      row_w.reshape(NT, 1, TILE))

    p = pos.reshape(T, TOPK)
    out = (jnp.take(ys, p[:, 0], axis=0).astype(jnp.float32)
           + jnp.take(ys, p[:, 1], axis=0).astype(jnp.float32))
    return out


# speedup vs baseline: 1.2634x; 1.2634x over previous
import functools

import jax
import jax.numpy as jnp
from jax import lax
from jax.experimental import pallas as pl
from jax.experimental.pallas import tpu as pltpu
from jax.experimental.pallas import tpu_sc as plsc

E = 8
TOPK = 2
T = 8192
D_MODEL = 2048
D_FF = 1408
BLOCK = 128

TILE = 256
N_PAD = T * TOPK + E * TILE
NT = N_PAD // TILE


def _dequant_body(w_ref, s_ref, o_ref):
    j = pl.program_id(1)
    w = w_ref[0]
    s = s_ref[0, pl.ds(j, 1), :][0]
    ncols = w.shape[1]
    wr = w.reshape(BLOCK, ncols // BLOCK, BLOCK)
    o = wr * s[None, :, None]
    o_ref[0] = o.reshape(BLOCK, ncols).astype(jnp.bfloat16)


def _dequant_to_bf16(w, s):
    _, n, k = w.shape
    return pl.pallas_call(
        _dequant_body,
        grid=(E, n // BLOCK),
        in_specs=[
            pl.BlockSpec((1, BLOCK, k), lambda e, j: (e, j, 0)),
            pl.BlockSpec((1, n // BLOCK, k // BLOCK), lambda e, j: (e, 0, 0)),
        ],
        out_specs=pl.BlockSpec((1, BLOCK, k), lambda e, j: (e, j, 0)),
        out_shape=jax.ShapeDtypeStruct(w.shape, jnp.bfloat16),
    )(w, s)


_NW = 32
_RPW = N_PAD // _NW
_CH = 24
_NCHUNK = _RPW // _CH
_SL = D_MODEL // 128


def _sc_gather_body(x_hbm, idx_hbm, out_hbm, idx_v, rows0, rows1, s0, s1):
    wid = lax.axis_index("s") * 2 + lax.axis_index("c")
    pltpu.sync_copy(idx_hbm.at[wid], idx_v)

    bufs = (rows0, rows1)
    sems = (s0, s1)
    copies = []
    for c in range(_NCHUNK):
        copies.append(pltpu.async_copy(
            x_hbm.at[idx_v.at[c]], bufs[c % 2], sems[c % 2]))
        if c >= 1:
            copies[c - 1].wait()
            pltpu.sync_copy(
                bufs[(c - 1) % 2],
                out_hbm.at[pl.ds(wid * _RPW + (c - 1) * _CH, _CH)])
    copies[-1].wait()
    pltpu.sync_copy(
        bufs[(_NCHUNK - 1) % 2],
        out_hbm.at[pl.ds(wid * _RPW + (_NCHUNK - 1) * _CH, _CH)])


def _sc_gather(x2, idx3):
    mesh = plsc.VectorSubcoreMesh(core_axis_name="c", subcore_axis_name="s")
    f = functools.partial(
        pl.kernel,
        mesh=mesh,
        out_type=jax.ShapeDtypeStruct((N_PAD, D_MODEL), jnp.float32),
        scratch_types=[
            pltpu.VMEM((_NCHUNK, _CH), jnp.int32),
            pltpu.VMEM((_CH, D_MODEL), jnp.float32),
            pltpu.VMEM((_CH, D_MODEL), jnp.float32),
            pltpu.SemaphoreType.DMA,
            pltpu.SemaphoreType.DMA,
        ],
    )(_sc_gather_body)
    return f(x2, idx3)


def _ffn_body(te_ref, xs_ref, w1_ref, w3_ref, w2_ref, rw_ref, ys_ref):
    i = pl.program_id(0)

    @pl.when(te_ref[i] < E)
    def _():
        xb = xs_ref[...].astype(jnp.bfloat16)
        dn = (((1,), (1,)), ((), ()))
        g = jax.lax.dot_general(xb, w1_ref[0], dn,
                                preferred_element_type=jnp.float32)
        u = jax.lax.dot_general(xb, w3_ref[0], dn,
                                preferred_element_type=jnp.float32)
        h = (g / (1.0 + jnp.exp(-g)) * u).astype(jnp.bfloat16)
        y = jax.lax.dot_general(h, w2_ref[0], dn,
                                preferred_element_type=jnp.float32)
        ys_ref[...] = (y * rw_ref[0, 0, :][:, None]).astype(jnp.bfloat16)


def _grouped_ffn(tile_expert, xs, w13b, w2b, row_w3d):
    grid_spec = pltpu.PrefetchScalarGridSpec(
        num_scalar_prefetch=1,
        grid=(NT,),
        in_specs=[
            pl.BlockSpec((TILE, D_MODEL), lambda i, te: (i, 0)),
            pl.BlockSpec((1, D_FF, D_MODEL),
                         lambda i, te: (jnp.minimum(te[i], E - 1), 0, 0)),
            pl.BlockSpec((1, D_FF, D_MODEL),
                         lambda i, te: (jnp.minimum(te[i], E - 1), 1, 0)),
            pl.BlockSpec((1, D_MODEL, D_FF),
                         lambda i, te: (jnp.minimum(te[i], E - 1), 0, 0)),
            pl.BlockSpec((1, 1, TILE), lambda i, te: (i, 0, 0)),
        ],
        out_specs=pl.BlockSpec((TILE, D_MODEL), lambda i, te: (i, 0)),
    )
    return pl.pallas_call(
        _ffn_body,
        grid_spec=grid_spec,
        out_shape=jax.ShapeDtypeStruct((N_PAD, D_MODEL), jnp.bfloat16),
        compiler_params=pltpu.CompilerParams(
            dimension_semantics=("arbitrary",)),
    )(tile_expert, xs, w13b, w13b, w2b, row_w3d)


def kernel(x, topk_ids, topk_weights, w13, w13_scale_inv, w2, w2_scale_inv):
    ids = topk_ids.reshape(-1).astype(jnp.int32)
    wflat = topk_weights.reshape(-1).astype(jnp.float32)
    tok = (jnp.arange(T * TOPK, dtype=jnp.int32) // TOPK)

    oh = (ids[:, None] == jnp.arange(E, dtype=jnp.int32)[None, :])
    oh3 = oh.astype(jnp.int32).reshape(128, T * TOPK // 128, E)
    win = jnp.cumsum(oh3, axis=1)
    chunk_tot = win[:, -1, :]
    chunk_off = jnp.cumsum(chunk_tot, axis=0) - chunk_tot
    rank3 = chunk_off[:, None, :] + win - oh3
    rank = jnp.sum(rank3 * oh3, axis=2).reshape(-1)
    counts = chunk_off[-1] + chunk_tot[-1]
    padded = ((counts + TILE - 1) // TILE) * TILE
    pad_off = jnp.cumsum(padded) - padded
    pos = pad_off[ids] + rank

    row_token = jnp.zeros((N_PAD,), jnp.int32).at[pos].set(tok)
    row_w = jnp.zeros((N_PAD,), jnp.float32).at[pos].set(wflat)

    tile_start = jnp.arange(NT, dtype=jnp.int32) * TILE
    pad_end = jnp.cumsum(padded)
    tile_expert = jnp.searchsorted(pad_end, tile_start, side="right")
    tile_expert = tile_expert.astype(jnp.int32)

    w13b = _dequant_to_bf16(w13, w13_scale_inv)
    w2b = _dequant_to_bf16(w2, w2_scale_inv)

    xs = _sc_gather(x, row_token.reshape(_NW, _NCHUNK, _CH))

    ys = _grouped_ffn(tile_expert, xs, w13b, w2b,
                      row_w.reshape(NT, 1, TILE))

    p = pos.reshape(T, TOPK)
    out = (jnp.take(ys, p[:, 0], axis=0).astype(jnp.float32)
           + jnp.take(ys, p[:, 1], axis=0).astype(jnp.float32))
    return out


# speedup vs baseline: 1.3125x; 1.0389x over previous
import functools

import jax
import jax.numpy as jnp
from jax import lax
from jax.experimental import pallas as pl
from jax.experimental.pallas import tpu as pltpu
from jax.experimental.pallas import tpu_sc as plsc

E = 8
TOPK = 2
T = 8192
D_MODEL = 2048
D_FF = 1408
BLOCK = 128

TILE = 256
N_PAD = T * TOPK + E * TILE
NT = N_PAD // TILE


def _dequant_body(w_ref, s_ref, o_ref):
    o_ref[0] = (w_ref[0] * s_ref[0, 0]).astype(jnp.bfloat16)


def _dequant_to_bf16(w, srow):
    _, n, k = w.shape
    return pl.pallas_call(
        _dequant_body,
        grid=(E, n // BLOCK),
        in_specs=[
            pl.BlockSpec((1, BLOCK, k), lambda e, j: (e, j, 0)),
            pl.BlockSpec((1, 1, 1, k), lambda e, j: (e, j, 0, 0)),
        ],
        out_specs=pl.BlockSpec((1, BLOCK, k), lambda e, j: (e, j, 0)),
        out_shape=jax.ShapeDtypeStruct(w.shape, jnp.bfloat16),
    )(w, srow)


def _expand_scales(s, k):
    return jnp.repeat(s, BLOCK, axis=2).reshape(s.shape[0], s.shape[1], 1, k)


_NW = 32
_RPW = N_PAD // _NW
_CH = 24
_NCHUNK = _RPW // _CH
_SL = D_MODEL // 128


def _sc_gather_body(x_hbm, idx_hbm, out_hbm, idx_v, rows0, rows1, s0, s1):
    wid = lax.axis_index("s") * 2 + lax.axis_index("c")
    pltpu.sync_copy(idx_hbm.at[wid], idx_v)

    bufs = (rows0, rows1)
    sems = (s0, s1)
    copies = []
    for c in range(_NCHUNK):
        copies.append(pltpu.async_copy(
            x_hbm.at[idx_v.at[c]], bufs[c % 2], sems[c % 2]))
        if c >= 1:
            copies[c - 1].wait()
            pltpu.sync_copy(
                bufs[(c - 1) % 2],
                out_hbm.at[pl.ds(wid * _RPW + (c - 1) * _CH, _CH)])
    copies[-1].wait()
    pltpu.sync_copy(
        bufs[(_NCHUNK - 1) % 2],
        out_hbm.at[pl.ds(wid * _RPW + (_NCHUNK - 1) * _CH, _CH)])


def _sc_gather(x2, idx3):
    mesh = plsc.VectorSubcoreMesh(core_axis_name="c", subcore_axis_name="s")
    f = functools.partial(
        pl.kernel,
        mesh=mesh,
        out_type=jax.ShapeDtypeStruct((N_PAD, D_MODEL), jnp.float32),
        scratch_types=[
            pltpu.VMEM((_NCHUNK, _CH), jnp.int32),
            pltpu.VMEM((_CH, D_MODEL), jnp.float32),
            pltpu.VMEM((_CH, D_MODEL), jnp.float32),
            pltpu.SemaphoreType.DMA,
            pltpu.SemaphoreType.DMA,
        ],
    )(_sc_gather_body)
    return f(x2, idx3)


def _ffn_body(te_ref, xs_ref, w1_ref, w3_ref, w2_ref, rw_ref, ys_ref):
    i = pl.program_id(0)

    @pl.when(te_ref[i] < E)
    def _():
        xb = xs_ref[...].astype(jnp.bfloat16)
        dn = (((1,), (1,)), ((), ()))
        g = jax.lax.dot_general(xb, w1_ref[0], dn,
                                preferred_element_type=jnp.float32)
        u = jax.lax.dot_general(xb, w3_ref[0], dn,
                                preferred_element_type=jnp.float32)
        h = (g / (1.0 + jnp.exp(-g)) * u).astype(jnp.bfloat16)
        y = jax.lax.dot_general(h, w2_ref[0], dn,
                                preferred_element_type=jnp.float32)
        ys_ref[...] = (y * rw_ref[0, 0, :][:, None]).astype(jnp.bfloat16)


def _grouped_ffn(tile_expert, xs, w13b, w2b, row_w3d):
    grid_spec = pltpu.PrefetchScalarGridSpec(
        num_scalar_prefetch=1,
        grid=(NT,),
        in_specs=[
            pl.BlockSpec((TILE, D_MODEL), lambda i, te: (i, 0)),
            pl.BlockSpec((1, D_FF, D_MODEL),
                         lambda i, te: (jnp.minimum(te[i], E - 1), 0, 0)),
            pl.BlockSpec((1, D_FF, D_MODEL),
                         lambda i, te: (jnp.minimum(te[i], E - 1), 1, 0)),
            pl.BlockSpec((1, D_MODEL, D_FF),
                         lambda i, te: (jnp.minimum(te[i], E - 1), 0, 0)),
            pl.BlockSpec((1, 1, TILE), lambda i, te: (i, 0, 0)),
        ],
        out_specs=pl.BlockSpec((TILE, D_MODEL), lambda i, te: (i, 0)),
    )
    return pl.pallas_call(
        _ffn_body,
        grid_spec=grid_spec,
        out_shape=jax.ShapeDtypeStruct((N_PAD, D_MODEL), jnp.bfloat16),
        compiler_params=pltpu.CompilerParams(
            dimension_semantics=("arbitrary",)),
    )(tile_expert, xs, w13b, w13b, w2b, row_w3d)


def kernel(x, topk_ids, topk_weights, w13, w13_scale_inv, w2, w2_scale_inv):
    ids = topk_ids.reshape(-1).astype(jnp.int32)
    wflat = topk_weights.reshape(-1).astype(jnp.float32)
    tok = (jnp.arange(T * TOPK, dtype=jnp.int32) // TOPK)

    oh = (ids[:, None] == jnp.arange(E, dtype=jnp.int32)[None, :])
    oh3 = oh.astype(jnp.int32).reshape(128, T * TOPK // 128, E)
    win = jnp.cumsum(oh3, axis=1)
    chunk_tot = win[:, -1, :]
    chunk_off = jnp.cumsum(chunk_tot, axis=0) - chunk_tot
    rank3 = chunk_off[:, None, :] + win - oh3
    rank = jnp.sum(rank3 * oh3, axis=2).reshape(-1)
    counts = chunk_off[-1] + chunk_tot[-1]
    padded = ((counts + TILE - 1) // TILE) * TILE
    pad_off = jnp.cumsum(padded) - padded
    pos = pad_off[ids] + rank

    row_token = jnp.zeros((N_PAD,), jnp.int32).at[pos].set(tok)
    row_w = jnp.zeros((N_PAD,), jnp.float32).at[pos].set(wflat)

    tile_start = jnp.arange(NT, dtype=jnp.int32) * TILE
    pad_end = jnp.cumsum(padded)
    tile_expert = jnp.searchsorted(pad_end, tile_start, side="right")
    tile_expert = tile_expert.astype(jnp.int32)

    w13b = _dequant_to_bf16(w13, _expand_scales(w13_scale_inv, D_MODEL))
    w2b = _dequant_to_bf16(w2, _expand_scales(w2_scale_inv, D_FF))

    xs = _sc_gather(x, row_token.reshape(_NW, _NCHUNK, _CH))

    ys = _grouped_ffn(tile_expert, xs, w13b, w2b,
                      row_w.reshape(NT, 1, TILE))

    p = pos.reshape(T, TOPK)
    out = (jnp.take(ys, p[:, 0], axis=0).astype(jnp.float32)
           + jnp.take(ys, p[:, 1], axis=0).astype(jnp.float32))
    return out
